# baseline (speedup 1.0000x reference)
import jax
import jax.numpy as jnp
from jax.experimental import pallas as pl


def _leaky(x):
    return jax.nn.leaky_relu(x, negative_slope=0.01)


def _segment_softmax(alpha, index, num_segments):
    amax = jax.ops.segment_max(alpha, index, num_segments=num_segments)
    amax = jnp.where(jnp.isfinite(amax), amax, 0.0)
    ex = jnp.exp(alpha - amax[index])
    denom = jax.ops.segment_sum(ex, index, num_segments=num_segments)
    return ex / (denom[index] + 1e-16)


def _gate_conv(x, edge_index, edge_attr, att_l, att_r, lin1_W, lin2_W, bias):
    src = edge_index[0]
    dst = edge_index[1]
    x_j = x[src]
    x_i = x[dst]
    h_j = _leaky(jnp.concatenate([x_j, edge_attr], axis=-1) @ lin1_W)
    alpha_j = jnp.sum(h_j * att_l, axis=-1)
    alpha_i = jnp.sum(x_i * att_r, axis=-1)
    alpha = _leaky(alpha_j + alpha_i)
    alpha = _segment_softmax(alpha, dst, x.shape[0])
    msg = (h_j @ lin2_W) * alpha[:, None]
    out = jax.ops.segment_sum(msg, dst, num_segments=x.shape[0])
    return out + bias


def _mlp_kernel(feat_ref, w1_ref, b1_ref, w2_ref, b2_ref, out_ref):
    f = feat_ref[...]
    h = jnp.maximum(f @ w1_ref[...] + b1_ref[...], 0.0)
    o = jax.nn.sigmoid(h @ w2_ref[...] + b2_ref[...])
    out_ref[...] = o[:, 0]


def kernel(x, edge_index, edge_attr, emb_table, att_l0, att_r0, lin1_W0, lin2_W0, bias0, att_l1, att_r1, lin1_W1, lin2_W1, bias1, fc1_W, fc1_b, fc2_W, fc2_b):
    h = emb_table[x]
    x_all = [h]
    layer_params = [
        (att_l0, att_r0, lin1_W0, lin2_W0, bias0),
        (att_l1, att_r1, lin1_W1, lin2_W1, bias1),
    ]
    cur = h
    for p in layer_params:
        cur = jax.nn.relu(_gate_conv(cur, edge_index, edge_attr, *p))
        x_all.append(cur)
    feat = jnp.concatenate(x_all, axis=-1)
    n = feat.shape[0]
    out = pl.pallas_call(
        _mlp_kernel,
        out_shape=jax.ShapeDtypeStruct((n,), jnp.float32),
    )(feat, fc1_W, fc1_b[None, :], fc2_W, fc2_b[None, :])
    return out


# speedup vs baseline: 4.0981x; 4.0981x over previous
import functools

import jax
import jax.numpy as jnp
from jax import lax
from jax.experimental import pallas as pl
from jax.experimental.pallas import tpu as pltpu
from jax.experimental.pallas import tpu_sc as plsc

N = 10000
NP = 10240
E = 320000
D = 128
NC, NS, LN = 2, 16, 16
NW = NC * NS
EW = E // NW
CH = 400
NCH = EW // CH
SUB = 80
NSUB = CH // SUB
NGRP = CH // LN
NT = NP // NS

_mesh = plsc.VectorSubcoreMesh(
    core_axis_name="c", subcore_axis_name="s", num_cores=NC, num_subcores=NS)


def _leaky(x):
    return jnp.where(x >= 0, x, 0.01 * x)


_GDN = lax.GatherDimensionNumbers(
    offset_dims=(), collapsed_slice_dims=(0,), start_index_map=(0,))


def _vgather(x, idx):
    return lax.gather(x, idx[:, None], _GDN, slice_sizes=(1,),
                      mode=lax.GatherScatterMode.PROMISE_IN_BOUNDS)


def _group_combine(key, val, op):
    lanes = lax.iota(jnp.int32, LN)
    acc = val
    for k in range(1, LN):
        idx = (lanes + k) & (LN - 1)
        kk = _vgather(key, idx)
        vv = _vgather(val, idx)
        eq = kk == key
        if op == "max":
            acc = jnp.where(eq, jnp.maximum(acc, vv), acc)
        else:
            acc = acc + jnp.where(eq, vv, 0.0)
    return acc


def _wid():
    return lax.axis_index("c") * NS + lax.axis_index("s")


def _vec_loop(ref_dst, n, fn):
    def body(i, _):
        ref_dst[pl.ds(i * LN, LN)] = fn(i)
        return 0
    lax.fori_loop(0, n // LN, body, 0)


@functools.partial(
    pl.kernel,
    out_type=jax.ShapeDtypeStruct((NP, D), jnp.float32),
    mesh=_mesh,
    compiler_params=pltpu.CompilerParams(needs_layout_passes=False, use_tc_tiling_on_sc=False),
    scratch_types=[
        pltpu.VMEM((NP // NW,), jnp.int32),
        pltpu.VMEM((NP // NW, D), jnp.float32),
        pltpu.SemaphoreType.DMA,
    ],
)
def _emb_gather(idx_hbm, tab_hbm, out_hbm, idx_v, rows_v, sem):
    bpw = NP // NW
    base = _wid() * bpw
    pltpu.sync_copy(idx_hbm.at[pl.ds(base, bpw)], idx_v)
    descs = [
        pltpu.async_copy(
            tab_hbm.at[idx_v.at[pl.ds(i * SUB, SUB)]],
            rows_v.at[pl.ds(i * SUB, SUB), :], sem)
        for i in range(bpw // SUB)
    ]
    for dsc in descs:
        dsc.wait()
    pltpu.sync_copy(rows_v, out_hbm.at[pl.ds(base, bpw), :])


@functools.partial(
    pl.kernel,
    out_type=jax.ShapeDtypeStruct((E, D), jnp.float32),
    mesh=_mesh,
    compiler_params=pltpu.CompilerParams(needs_layout_passes=False, use_tc_tiling_on_sc=False),
    scratch_types=[
        pltpu.VMEM((CH,), jnp.int32),
        pltpu.VMEM((CH, D), jnp.float32),
        pltpu.SemaphoreType.DMA,
    ],
)
def _edge_gather(src_hbm, xl1_hbm, g_hbm, idx_v, rows_v, sem):
    ebase = _wid() * EW

    def chunk(c, _):
        b = ebase + c * CH
        pltpu.sync_copy(src_hbm.at[pl.ds(b, CH)], idx_v)
        descs = [
            pltpu.async_copy(
                xl1_hbm.at[idx_v.at[pl.ds(i * SUB, SUB)]],
                rows_v.at[pl.ds(i * SUB, SUB), :], sem)
            for i in range(NSUB)
        ]
        for dsc in descs:
            dsc.wait()
        pltpu.sync_copy(rows_v, g_hbm.at[pl.ds(b, CH), :])
        return 0

    lax.fori_loop(0, NCH, chunk, 0)


@functools.partial(
    pl.kernel,
    out_type=(
        jax.ShapeDtypeStruct((E,), jnp.float32),
        jax.ShapeDtypeStruct((NC, NP), jnp.float32),
        jax.ShapeDtypeStruct((NC, NS, NP), jnp.float32),
    ),
    mesh=_mesh,
    compiler_params=pltpu.CompilerParams(needs_layout_passes=False, use_tc_tiling_on_sc=False),
    scratch_types=[
        pltpu.VMEM((NP,), jnp.float32),
        pltpu.VMEM((NP,), jnp.float32),
        pltpu.VMEM((CH,), jnp.int32),
        pltpu.VMEM((CH,), jnp.float32),
        pltpu.VMEM((CH,), jnp.float32),
        pltpu.VMEM((NT,), jnp.float32),
        pltpu.VMEM((NT,), jnp.float32),
    ],
)
def _alpha_max(aj_hbm, dst_hbm, ar_hbm, a_hbm, mpart_hbm, m_all_hbm,
               ar_full, m_tile, dst_v, aj_v, a_v, acc_v, tmp_v):
    c = lax.axis_index("c")
    s = lax.axis_index("s")
    ebase = (c * NS + s) * EW
    pltpu.sync_copy(ar_hbm, ar_full)
    _vec_loop(m_tile, NP, lambda i: jnp.full((LN,), -1e30, jnp.float32))

    def chunk(cix, _):
        b = ebase + cix * CH
        pltpu.sync_copy(dst_hbm.at[pl.ds(b, CH)], dst_v)
        pltpu.sync_copy(aj_hbm.at[pl.ds(b, CH)], aj_v)

        def grp(g, _):
            d16 = dst_v[pl.ds(g * LN, LN)]
            aj16 = aj_v[pl.ds(g * LN, LN)]
            a16 = _leaky(aj16 + plsc.load_gather(ar_full, [d16]))
            a_v[pl.ds(g * LN, LN)] = a16
            gm = _group_combine(d16, a16, "max")
            old = plsc.load_gather(m_tile, [d16])
            plsc.store_scatter(m_tile, [d16], jnp.maximum(old, gm))
            return 0

        lax.fori_loop(0, NGRP, grp, 0)
        pltpu.sync_copy(a_v, a_hbm.at[pl.ds(b, CH)])
        return 0

    lax.fori_loop(0, NCH, chunk, 0)

    pltpu.sync_copy(m_tile, m_all_hbm.at[c, s])
    plsc.subcore_barrier()
    nb = s * NT
    pltpu.sync_copy(m_all_hbm.at[c, 0, pl.ds(nb, NT)], acc_v)

    def comb(j, _):
        pltpu.sync_copy(m_all_hbm.at[c, j, pl.ds(nb, NT)], tmp_v)
        _vec_loop(acc_v, NT, lambda i: jnp.maximum(
            acc_v[pl.ds(i * LN, LN)], tmp_v[pl.ds(i * LN, LN)]))
        return 0

    lax.fori_loop(1, NS, comb, 0)
    pltpu.sync_copy(acc_v, mpart_hbm.at[c, pl.ds(nb, NT)])


@functools.partial(
    pl.kernel,
    out_type=(
        jax.ShapeDtypeStruct((E,), jnp.float32),
        jax.ShapeDtypeStruct((NC, NP), jnp.float32),
        jax.ShapeDtypeStruct((NC, NS, NP), jnp.float32),
    ),
    mesh=_mesh,
    compiler_params=pltpu.CompilerParams(needs_layout_passes=False, use_tc_tiling_on_sc=False),
    scratch_types=[
        pltpu.VMEM((NP,), jnp.float32),
        pltpu.VMEM((NP,), jnp.float32),
        pltpu.VMEM((NP,), jnp.float32),
        pltpu.VMEM((CH,), jnp.int32),
        pltpu.VMEM((CH,), jnp.float32),
        pltpu.VMEM((CH,), jnp.float32),
        pltpu.VMEM((NT,), jnp.float32),
        pltpu.VMEM((NT,), jnp.float32),
    ],
)
def _exp_den(a_hbm, dst_hbm, mpart_hbm, ex_hbm, denpart_hbm, d_all_hbm,
             m_full, tmp_full, den_tile, dst_v, a_v, ex_v, acc_v, tmp_v):
    c = lax.axis_index("c")
    s = lax.axis_index("s")
    ebase = (c * NS + s) * EW
    pltpu.sync_copy(mpart_hbm.at[0], m_full)
    pltpu.sync_copy(mpart_hbm.at[1], tmp_full)
    _vec_loop(m_full, NP, lambda i: jnp.maximum(
        m_full[pl.ds(i * LN, LN)], tmp_full[pl.ds(i * LN, LN)]))
    _vec_loop(den_tile, NP, lambda i: jnp.zeros((LN,), jnp.float32))

    def chunk(cix, _):
        b = ebase + cix * CH
        pltpu.sync_copy(dst_hbm.at[pl.ds(b, CH)], dst_v)
        pltpu.sync_copy(a_hbm.at[pl.ds(b, CH)], a_v)

        def grp(g, _):
            d16 = dst_v[pl.ds(g * LN, LN)]
            a16 = a_v[pl.ds(g * LN, LN)]
            ex16 = jnp.exp(a16 - plsc.load_gather(m_full, [d16]))
            ex_v[pl.ds(g * LN, LN)] = ex16
            gs = _group_combine(d16, ex16, "sum")
            old = plsc.load_gather(den_tile, [d16])
            plsc.store_scatter(den_tile, [d16], old + gs)
            return 0

        lax.fori_loop(0, NGRP, grp, 0)
        pltpu.sync_copy(ex_v, ex_hbm.at[pl.ds(b, CH)])
        return 0

    lax.fori_loop(0, NCH, chunk, 0)

    pltpu.sync_copy(den_tile, d_all_hbm.at[c, s])
    plsc.subcore_barrier()
    nb = s * NT
    pltpu.sync_copy(d_all_hbm.at[c, 0, pl.ds(nb, NT)], acc_v)

    def comb(j, _):
        pltpu.sync_copy(d_all_hbm.at[c, j, pl.ds(nb, NT)], tmp_v)
        _vec_loop(acc_v, NT, lambda i: acc_v[pl.ds(i * LN, LN)] +
                  tmp_v[pl.ds(i * LN, LN)])
        return 0

    lax.fori_loop(1, NS, comb, 0)
    pltpu.sync_copy(acc_v, denpart_hbm.at[c, pl.ds(nb, NT)])


_ZR = 160


HD = D // NC
EW5 = E // NS
NCH5 = EW5 // CH


@functools.partial(
    pl.kernel,
    out_type=jax.ShapeDtypeStruct((NC, NP, HD), jnp.float32),
    mesh=_mesh,
    compiler_params=pltpu.CompilerParams(needs_layout_passes=False, use_tc_tiling_on_sc=False),
    scratch_types=[
        pltpu.VMEM((NP,), jnp.float32),
        pltpu.VMEM((NP,), jnp.float32),
        pltpu.VMEM((CH,), jnp.int32),
        pltpu.VMEM((NSUB, SUB), jnp.int32),
        pltpu.VMEM((CH,), jnp.float32),
        pltpu.VMEM((CH,), jnp.float32),
        pltpu.VMEM((CH, HD), jnp.float32),
        pltpu.VMEM((_ZR, HD), jnp.float32),
        pltpu.VMEM_SHARED((NP, HD), jnp.float32),
        pltpu.SemaphoreType.DMA,
    ],
)
def _aggregate(h_hbm, ex_hbm, dst_hbm, denpart_hbm, spart_hbm,
               dinv, tmp_d, dst_v, dst2d, ex_v, w_v, rows_v, zero_v, s_spmem, sem):
    c = lax.axis_index("c")
    s = lax.axis_index("s")
    ebase = s * EW5
    pltpu.sync_copy(denpart_hbm.at[0], dinv)
    pltpu.sync_copy(denpart_hbm.at[1], tmp_d)
    _vec_loop(dinv, NP, lambda i: 1.0 / (
        dinv[pl.ds(i * LN, LN)] + tmp_d[pl.ds(i * LN, LN)] + 1e-16))

    def zrow16(i, _):
        for j in range(HD // LN):
            zero_v[i, pl.ds(j * LN, LN)] = jnp.zeros((LN,), jnp.float32)
        return 0
    lax.fori_loop(0, _ZR, zrow16, 0)
    nb = s * NT
    for q in range(NT // _ZR):
        pltpu.sync_copy(zero_v, s_spmem.at[pl.ds(nb + q * _ZR, _ZR), :])
    plsc.subcore_barrier()

    def chunk(cix, _):
        b = ebase + cix * CH
        pltpu.sync_copy(dst_hbm.at[pl.ds(b, CH)], dst_v)
        for i in range(NSUB):
            pltpu.sync_copy(dst_hbm.at[pl.ds(b + i * SUB, SUB)], dst2d.at[i])
        pltpu.sync_copy(ex_hbm.at[pl.ds(b, CH)], ex_v)
        pltpu.sync_copy(h_hbm.at[pl.ds(b, CH), pl.ds(c * HD, HD)], rows_v)

        def grp(g, _):
            d16 = dst_v[pl.ds(g * LN, LN)]
            w_v[pl.ds(g * LN, LN)] = (
                ex_v[pl.ds(g * LN, LN)] * plsc.load_gather(dinv, [d16]))
            return 0

        lax.fori_loop(0, NGRP, grp, 0)

        def scale(g, _):
            w16 = w_v[pl.ds(g * LN, LN)]
            for le in range(LN):
                e = g * LN + le
                we = w16[le]
                for j in range(HD // LN):
                    rows_v[e, pl.ds(j * LN, LN)] = (
                        rows_v[e, pl.ds(j * LN, LN)] * we)
            return 0

        lax.fori_loop(0, NGRP, scale, 0)
        for i in range(NSUB):
            pltpu.sync_copy(
                rows_v.at[pl.ds(i * SUB, SUB), :],
                s_spmem.at[dst2d.at[i]], add=True)
        return 0

    lax.fori_loop(0, NCH5, chunk, 0)
    plsc.subcore_barrier()
    for q in range(NT // _ZR):
        pltpu.sync_copy(
            s_spmem.at[pl.ds(nb + q * _ZR, _ZR), :],
            spart_hbm.at[c, pl.ds(nb + q * _ZR, _ZR), :])


def _node_dense0_body(cur_ref, w1x_ref, attr_ref, xl1_ref, ar_ref):
    cur = cur_ref[...]
    xl1_ref[...] = jnp.dot(cur, w1x_ref[...], preferred_element_type=jnp.float32)
    ar_ref[...] = jnp.sum(cur * attr_ref[...], axis=1).reshape(1, 4, 128)


def _node_dense_body(sp_ref, w2_ref, b_ref, w1x_ref, attr_ref,
                     cur_ref, xl1_ref, ar_ref):
    sacc = jnp.concatenate([sp_ref[0], sp_ref[1]], axis=1)
    cur = jnp.maximum(
        jnp.dot(sacc, w2_ref[...], preferred_element_type=jnp.float32)
        + b_ref[...], 0.0)
    cur_ref[...] = cur
    xl1_ref[...] = jnp.dot(cur, w1x_ref[...], preferred_element_type=jnp.float32)
    ar_ref[...] = jnp.sum(cur * attr_ref[...], axis=1).reshape(1, 4, 128)


def _edge_dense_body(g_ref, ea_ref, w1e_ref, attl_ref, h_ref, aj_ref):
    z = g_ref[...] + jnp.dot(ea_ref[...], w1e_ref[...],
                             preferred_element_type=jnp.float32)
    h = jnp.where(z >= 0, z, 0.01 * z)
    h_ref[...] = h
    aj_ref[...] = jnp.sum(h * attl_ref[...], axis=1).reshape(1, 4, 128)


def _final_body(sp_ref, w2_ref, b_ref, h_ref, c1_ref,
                f1h_ref, f1c1_ref, f1c2_ref, f1b_ref, f2w_ref, f2b_ref, out_ref):
    sacc = jnp.concatenate([sp_ref[0], sp_ref[1]], axis=1)
    c2 = jnp.maximum(
        jnp.dot(sacc, w2_ref[...], preferred_element_type=jnp.float32)
        + b_ref[...], 0.0)
    f = (jnp.dot(h_ref[...], f1h_ref[...], preferred_element_type=jnp.float32)
         + jnp.dot(c1_ref[...], f1c1_ref[...], preferred_element_type=jnp.float32)
         + jnp.dot(c2, f1c2_ref[...], preferred_element_type=jnp.float32)
         + f1b_ref[...])
    f = jnp.maximum(f, 0.0)
    o = jnp.dot(f, f2w_ref[...], preferred_element_type=jnp.float32) + f2b_ref[...]
    out_ref[...] = jax.nn.sigmoid(o[:, 0]).reshape(1, 4, 128)


_NB = 512
_NG = NP // _NB
_EB = 512
_EG = E // _EB

_full = lambda shape: pl.BlockSpec(shape, lambda i: tuple(0 for _ in shape))
_blk = lambda shape: pl.BlockSpec(shape, lambda i: (i,) + tuple(0 for _ in shape[1:]))


def _node_dense0(cur, w1x, att_r):
    return pl.pallas_call(
        _node_dense0_body,
        grid=(_NG,),
        in_specs=[_blk((_NB, D)), _full((D, D)), _full((1, D))],
        out_specs=[_blk((_NB, D)), _blk((1, 4, 128))],
        out_shape=[
            jax.ShapeDtypeStruct((NP, D), jnp.float32),
            jax.ShapeDtypeStruct((_NG, 4, 128), jnp.float32),
        ],
    )(cur, w1x, att_r.reshape(1, D))


def _node_dense(spart, w2, bias, w1x, att_r):
    return pl.pallas_call(
        _node_dense_body,
        grid=(_NG,),
        in_specs=[
            pl.BlockSpec((NC, _NB, HD), lambda i: (0, i, 0)),
            _full((D, D)), _full((1, D)), _full((D, D)), _full((1, D)),
        ],
        out_specs=[_blk((_NB, D)), _blk((_NB, D)), _blk((1, 4, 128))],
        out_shape=[
            jax.ShapeDtypeStruct((NP, D), jnp.float32),
            jax.ShapeDtypeStruct((NP, D), jnp.float32),
            jax.ShapeDtypeStruct((_NG, 4, 128), jnp.float32),
        ],
    )(spart, w2, bias.reshape(1, D), w1x, att_r.reshape(1, D))


def _edge_dense(g, ea8, w1e8, att_l):
    return pl.pallas_call(
        _edge_dense_body,
        grid=(_EG,),
        in_specs=[_blk((_EB, D)), _blk((_EB, 8)), _full((8, D)), _full((1, D))],
        out_specs=[_blk((_EB, D)), _blk((1, 4, 128))],
        out_shape=[
            jax.ShapeDtypeStruct((E, D), jnp.float32),
            jax.ShapeDtypeStruct((_EG, 4, 128), jnp.float32),
        ],
    )(g, ea8, w1e8, att_l.reshape(1, D))


def _final(spart, w2, bias, h, c1, f1h, f1c1, f1c2, f1b, f2w, f2b):
    return pl.pallas_call(
        _final_body,
        grid=(_NG,),
        in_specs=[
            pl.BlockSpec((NC, _NB, HD), lambda i: (0, i, 0)),
            _full((D, D)), _full((1, D)),
            _blk((_NB, D)), _blk((_NB, D)),
            _full((D, 128)), _full((D, 128)), _full((D, 128)),
            _full((1, 128)), _full((128, 128)), _full((1, 128)),
        ],
        out_specs=[_blk((1, 4, 128))],
        out_shape=[jax.ShapeDtypeStruct((_NG, 4, 128), jnp.float32)],
    )(spart, w2, bias.reshape(1, D), h, c1, f1h, f1c1, f1c2, f1b, f2w, f2b)


def kernel(x, edge_index, edge_attr, emb_table,
           att_l0, att_r0, lin1_W0, lin2_W0, bias0,
           att_l1, att_r1, lin1_W1, lin2_W1, bias1,
           fc1_W, fc1_b, fc2_W, fc2_b):
    src = edge_index[0]
    dst = edge_index[1]
    x_pad = jnp.pad(x, (0, NP - N))
    ea8 = jnp.pad(edge_attr, ((0, 0), (0, 1)))

    h0 = _emb_gather(x_pad, emb_table)

    def layer(cur_args, att_l, att_r, lin1_W, lin2_W, bias, first):
        w1x = lin1_W[:D]
        w1e8 = jnp.pad(lin1_W[D:], ((0, 1), (0, 0)))
        if first:
            cur = cur_args
            xl1, ar2 = _node_dense0(cur, w1x, att_r)
        else:
            spart, w2p, bp = cur_args
            cur, xl1, ar2 = _node_dense(spart, w2p, bp, w1x, att_r)
        ar = ar2.reshape(NP)
        g = _edge_gather(src, xl1)
        hh, aj2 = _edge_dense(g, ea8, w1e8, att_l)
        aj = aj2.reshape(E)
        a, mpart, _ = _alpha_max(aj, dst, ar)
        ex, denpart, _ = _exp_den(a, dst, mpart)
        spart = _aggregate(hh, ex, dst, denpart)
        return cur, spart

    _, spart0 = layer(h0, att_l0, att_r0, lin1_W0, lin2_W0, bias0, True)
    cur1, spart1 = layer((spart0, lin2_W0, bias0),
                         att_l1, att_r1, lin1_W1, lin2_W1, bias1, False)

    f1h = jnp.pad(fc1_W[:D], ((0, 0), (0, 108)))
    f1c1 = jnp.pad(fc1_W[D:2 * D], ((0, 0), (0, 108)))
    f1c2 = jnp.pad(fc1_W[2 * D:], ((0, 0), (0, 108)))
    f1b = jnp.pad(fc1_b, (0, 108)).reshape(1, 128)
    f2w = jnp.pad(fc2_W, ((0, 108), (0, 127)))
    f2b = jnp.pad(fc2_b, (0, 127)).reshape(1, 128)

    [out2] = _final(spart1, lin2_W1, bias1, h0, cur1,
                  f1h, f1c1, f1c2, f1b, f2w, f2b)
    return out2.reshape(NP)[:N]


# speedup vs baseline: 4.1372x; 1.0095x over previous
import functools

import jax
import jax.numpy as jnp
from jax import lax
from jax.experimental import pallas as pl
from jax.experimental.pallas import tpu as pltpu
from jax.experimental.pallas import tpu_sc as plsc

N = 10000
NP = 10240
E = 320000
D = 128
NC, NS, LN = 2, 16, 16
NW = NC * NS
EW = E // NW
CH = 400
NCH = EW // CH
SUB = 80
NSUB = CH // SUB
NGRP = CH // LN
NT = NP // NS

_mesh = plsc.VectorSubcoreMesh(
    core_axis_name="c", subcore_axis_name="s", num_cores=NC, num_subcores=NS)


def _leaky(x):
    return jnp.where(x >= 0, x, 0.01 * x)


_GDN = lax.GatherDimensionNumbers(
    offset_dims=(), collapsed_slice_dims=(0,), start_index_map=(0,))


def _vgather(x, idx):
    return lax.gather(x, idx[:, None], _GDN, slice_sizes=(1,),
                      mode=lax.GatherScatterMode.PROMISE_IN_BOUNDS)


def _group_combine(key, val, op):
    lanes = lax.iota(jnp.int32, LN)
    acc = val
    for k in range(1, LN):
        idx = (lanes + k) & (LN - 1)
        kk = _vgather(key, idx)
        vv = _vgather(val, idx)
        eq = kk == key
        if op == "max":
            acc = jnp.where(eq, jnp.maximum(acc, vv), acc)
        else:
            acc = acc + jnp.where(eq, vv, 0.0)
    return acc


def _wid():
    return lax.axis_index("c") * NS + lax.axis_index("s")


def _vec_loop(ref_dst, n, fn):
    def body(i, _):
        ref_dst[pl.ds(i * LN, LN)] = fn(i)
        return 0
    lax.fori_loop(0, n // LN, body, 0)


@functools.partial(
    pl.kernel,
    out_type=jax.ShapeDtypeStruct((NP, D), jnp.float32),
    mesh=_mesh,
    compiler_params=pltpu.CompilerParams(needs_layout_passes=False, use_tc_tiling_on_sc=False),
    scratch_types=[
        pltpu.VMEM((NP // NW,), jnp.int32),
        pltpu.VMEM((NP // NW, D), jnp.float32),
        pltpu.SemaphoreType.DMA,
    ],
)
def _emb_gather(idx_hbm, tab_hbm, out_hbm, idx_v, rows_v, sem):
    bpw = NP // NW
    base = _wid() * bpw
    pltpu.sync_copy(idx_hbm.at[pl.ds(base, bpw)], idx_v)
    descs = [
        pltpu.async_copy(
            tab_hbm.at[idx_v.at[pl.ds(i * SUB, SUB)]],
            rows_v.at[pl.ds(i * SUB, SUB), :], sem)
        for i in range(bpw // SUB)
    ]
    for dsc in descs:
        dsc.wait()
    pltpu.sync_copy(rows_v, out_hbm.at[pl.ds(base, bpw), :])


@functools.partial(
    pl.kernel,
    out_type=jax.ShapeDtypeStruct((E, D), jnp.float32),
    mesh=_mesh,
    compiler_params=pltpu.CompilerParams(needs_layout_passes=False, use_tc_tiling_on_sc=False),
    scratch_types=[
        pltpu.VMEM((CH,), jnp.int32),
        pltpu.VMEM((CH, D), jnp.float32),
        pltpu.SemaphoreType.DMA,
    ],
)
def _edge_gather(src_hbm, xl1_hbm, g_hbm, idx_v, rows_v, sem):
    ebase = _wid() * EW

    def chunk(c, _):
        b = ebase + c * CH
        pltpu.sync_copy(src_hbm.at[pl.ds(b, CH)], idx_v)
        descs = [
            pltpu.async_copy(
                xl1_hbm.at[idx_v.at[pl.ds(i * SUB, SUB)]],
                rows_v.at[pl.ds(i * SUB, SUB), :], sem)
            for i in range(NSUB)
        ]
        for dsc in descs:
            dsc.wait()
        pltpu.sync_copy(rows_v, g_hbm.at[pl.ds(b, CH), :])
        return 0

    lax.fori_loop(0, NCH, chunk, 0)


@functools.partial(
    pl.kernel,
    out_type=(
        jax.ShapeDtypeStruct((E,), jnp.float32),
        jax.ShapeDtypeStruct((NW, LN), jnp.float32),
    ),
    mesh=_mesh,
    compiler_params=pltpu.CompilerParams(needs_layout_passes=False, use_tc_tiling_on_sc=False),
    scratch_types=[
        pltpu.VMEM((NP,), jnp.float32),
        pltpu.VMEM((CH,), jnp.int32),
        pltpu.VMEM((CH,), jnp.float32),
        pltpu.VMEM((CH,), jnp.float32),
        pltpu.VMEM((LN,), jnp.float32),
    ],
)
def _alpha_max(aj_hbm, dst_hbm, ar_hbm, a_hbm, mx_hbm,
               ar_full, dst_v, aj_v, a_v, mx_v):
    w = _wid()
    ebase = w * EW
    pltpu.sync_copy(ar_hbm, ar_full)

    def chunk(cix, mx):
        b = ebase + cix * CH
        pltpu.sync_copy(dst_hbm.at[pl.ds(b, CH)], dst_v)
        pltpu.sync_copy(aj_hbm.at[pl.ds(b, CH)], aj_v)

        def grp(g, mxg):
            d16 = dst_v[pl.ds(g * LN, LN)]
            aj16 = aj_v[pl.ds(g * LN, LN)]
            a16 = _leaky(aj16 + plsc.load_gather(ar_full, [d16]))
            a_v[pl.ds(g * LN, LN)] = a16
            return jnp.maximum(mxg, a16)

        mx = lax.fori_loop(0, NGRP, grp, mx)
        pltpu.sync_copy(a_v, a_hbm.at[pl.ds(b, CH)])
        return mx

    mx = lax.fori_loop(0, NCH, chunk, jnp.full((LN,), -1e30, jnp.float32))
    mx_v[...] = mx
    pltpu.sync_copy(mx_v, mx_hbm.at[w])


@functools.partial(
    pl.kernel,
    out_type=(
        jax.ShapeDtypeStruct((E,), jnp.float32),
        jax.ShapeDtypeStruct((NC, NP), jnp.float32),
    ),
    mesh=_mesh,
    compiler_params=pltpu.CompilerParams(needs_layout_passes=False, use_tc_tiling_on_sc=False),
    scratch_types=[
        pltpu.VMEM((NW, LN), jnp.float32),
        pltpu.VMEM((CH,), jnp.float32),
        pltpu.VMEM((NSUB, SUB), jnp.int32),
        pltpu.VMEM((NSUB, SUB), jnp.float32),
        pltpu.VMEM((NT,), jnp.float32),
        pltpu.VMEM_SHARED((NP,), jnp.float32),
    ],
)
def _exp_den(a_hbm, dst_hbm, mx_hbm, ex_hbm, denpart_hbm,
             mall_v, a_v, dst2d, ex2d, zb_v, den_spmem):
    c = lax.axis_index("c")
    s = lax.axis_index("s")
    ebase = (c * NS + s) * EW
    pltpu.sync_copy(mx_hbm, mall_v)
    acc = mall_v[0, :]
    for i in range(1, NW):
        acc = jnp.maximum(acc, mall_v[i, :])
    M = jnp.max(acc)

    _vec_loop(zb_v, NT, lambda i: jnp.zeros((LN,), jnp.float32))
    nb = s * NT
    pltpu.sync_copy(zb_v, den_spmem.at[pl.ds(nb, NT)])
    plsc.subcore_barrier()

    def chunk(cix, _):
        b = ebase + cix * CH
        pltpu.sync_copy(a_hbm.at[pl.ds(b, CH)], a_v)
        for i in range(NSUB):
            pltpu.sync_copy(dst_hbm.at[pl.ds(b + i * SUB, SUB)], dst2d.at[i])
        for g in range(NGRP):
            i, r = divmod(g, SUB // LN)
            ex16 = jnp.exp(a_v[pl.ds(g * LN, LN)] - M)
            ex2d[i, pl.ds(r * LN, LN)] = ex16
        for i in range(NSUB):
            pltpu.sync_copy(ex2d.at[i], ex_hbm.at[pl.ds(b + i * SUB, SUB)])
            pltpu.sync_copy(ex2d.at[i], den_spmem.at[dst2d.at[i]], add=True)
        return 0

    lax.fori_loop(0, NCH, chunk, 0)
    plsc.subcore_barrier()
    pltpu.sync_copy(den_spmem.at[pl.ds(nb, NT)],
                    denpart_hbm.at[c, pl.ds(nb, NT)])


_ZR = 160


HD = D // NC
EW5 = E // NS
NCH5 = EW5 // CH


@functools.partial(
    pl.kernel,
    out_type=jax.ShapeDtypeStruct((NC, NP, HD), jnp.float32),
    mesh=_mesh,
    compiler_params=pltpu.CompilerParams(needs_layout_passes=False, use_tc_tiling_on_sc=False),
    scratch_types=[
        pltpu.VMEM((NP,), jnp.float32),
        pltpu.VMEM((NP,), jnp.float32),
        pltpu.VMEM((CH,), jnp.int32),
        pltpu.VMEM((NSUB, SUB), jnp.int32),
        pltpu.VMEM((CH,), jnp.float32),
        pltpu.VMEM((CH,), jnp.float32),
        pltpu.VMEM((CH, HD), jnp.float32),
        pltpu.VMEM((_ZR, HD), jnp.float32),
        pltpu.VMEM_SHARED((NP, HD), jnp.float32),
        pltpu.SemaphoreType.DMA,
    ],
)
def _aggregate(h_hbm, ex_hbm, dst_hbm, denpart_hbm, spart_hbm,
               dinv, tmp_d, dst_v, dst2d, ex_v, w_v, rows_v, zero_v, s_spmem, sem):
    c = lax.axis_index("c")
    s = lax.axis_index("s")
    ebase = s * EW5
    pltpu.sync_copy(denpart_hbm.at[0], dinv)
    pltpu.sync_copy(denpart_hbm.at[1], tmp_d)
    _vec_loop(dinv, NP, lambda i: 1.0 / (
        dinv[pl.ds(i * LN, LN)] + tmp_d[pl.ds(i * LN, LN)] + 1e-16))

    def zrow16(i, _):
        for j in range(HD // LN):
            zero_v[i, pl.ds(j * LN, LN)] = jnp.zeros((LN,), jnp.float32)
        return 0
    lax.fori_loop(0, _ZR, zrow16, 0)
    nb = s * NT
    for q in range(NT // _ZR):
        pltpu.sync_copy(zero_v, s_spmem.at[pl.ds(nb + q * _ZR, _ZR), :])
    plsc.subcore_barrier()

    def chunk(cix, _):
        b = ebase + cix * CH
        pltpu.sync_copy(dst_hbm.at[pl.ds(b, CH)], dst_v)
        for i in range(NSUB):
            pltpu.sync_copy(dst_hbm.at[pl.ds(b + i * SUB, SUB)], dst2d.at[i])
        pltpu.sync_copy(ex_hbm.at[pl.ds(b, CH)], ex_v)
        pltpu.sync_copy(h_hbm.at[pl.ds(b, CH), pl.ds(c * HD, HD)], rows_v)

        def grp(g, _):
            d16 = dst_v[pl.ds(g * LN, LN)]
            w_v[pl.ds(g * LN, LN)] = (
                ex_v[pl.ds(g * LN, LN)] * plsc.load_gather(dinv, [d16]))
            return 0

        lax.fori_loop(0, NGRP, grp, 0)

        def scale(g, _):
            w16 = w_v[pl.ds(g * LN, LN)]
            for le in range(LN):
                e = g * LN + le
                we = w16[le]
                for j in range(HD // LN):
                    rows_v[e, pl.ds(j * LN, LN)] = (
                        rows_v[e, pl.ds(j * LN, LN)] * we)
            return 0

        lax.fori_loop(0, NGRP, scale, 0)
        for i in range(NSUB):
            pltpu.sync_copy(
                rows_v.at[pl.ds(i * SUB, SUB), :],
                s_spmem.at[dst2d.at[i]], add=True)
        return 0

    lax.fori_loop(0, NCH5, chunk, 0)
    plsc.subcore_barrier()
    for q in range(NT // _ZR):
        pltpu.sync_copy(
            s_spmem.at[pl.ds(nb + q * _ZR, _ZR), :],
            spart_hbm.at[c, pl.ds(nb + q * _ZR, _ZR), :])


def _node_dense0_body(cur_ref, w1x_ref, attr_ref, xl1_ref, ar_ref):
    cur = cur_ref[...]
    xl1_ref[...] = jnp.dot(cur, w1x_ref[...], preferred_element_type=jnp.float32)
    ar_ref[...] = jnp.sum(cur * attr_ref[...], axis=1).reshape(1, 4, 128)


def _node_dense_body(sp_ref, w2_ref, b_ref, w1x_ref, attr_ref,
                     cur_ref, xl1_ref, ar_ref):
    sacc = jnp.concatenate([sp_ref[0], sp_ref[1]], axis=1)
    cur = jnp.maximum(
        jnp.dot(sacc, w2_ref[...], preferred_element_type=jnp.float32)
        + b_ref[...], 0.0)
    cur_ref[...] = cur
    xl1_ref[...] = jnp.dot(cur, w1x_ref[...], preferred_element_type=jnp.float32)
    ar_ref[...] = jnp.sum(cur * attr_ref[...], axis=1).reshape(1, 4, 128)


def _edge_dense_body(g_ref, ea_ref, w1e_ref, attl_ref, h_ref, aj_ref):
    z = g_ref[...] + jnp.dot(ea_ref[...], w1e_ref[...],
                             preferred_element_type=jnp.float32)
    h = jnp.where(z >= 0, z, 0.01 * z)
    h_ref[...] = h
    aj_ref[...] = jnp.sum(h * attl_ref[...], axis=1).reshape(1, 4, 128)


def _final_body(sp_ref, w2_ref, b_ref, h_ref, c1_ref,
                f1h_ref, f1c1_ref, f1c2_ref, f1b_ref, f2w_ref, f2b_ref, out_ref):
    sacc = jnp.concatenate([sp_ref[0], sp_ref[1]], axis=1)
    c2 = jnp.maximum(
        jnp.dot(sacc, w2_ref[...], preferred_element_type=jnp.float32)
        + b_ref[...], 0.0)
    f = (jnp.dot(h_ref[...], f1h_ref[...], preferred_element_type=jnp.float32)
         + jnp.dot(c1_ref[...], f1c1_ref[...], preferred_element_type=jnp.float32)
         + jnp.dot(c2, f1c2_ref[...], preferred_element_type=jnp.float32)
         + f1b_ref[...])
    f = jnp.maximum(f, 0.0)
    o = jnp.dot(f, f2w_ref[...], preferred_element_type=jnp.float32) + f2b_ref[...]
    out_ref[...] = jax.nn.sigmoid(o[:, 0]).reshape(1, 4, 128)


_NB = 512
_NG = NP // _NB
_EB = 512
_EG = E // _EB

_full = lambda shape: pl.BlockSpec(shape, lambda i: tuple(0 for _ in shape))
_blk = lambda shape: pl.BlockSpec(shape, lambda i: (i,) + tuple(0 for _ in shape[1:]))


def _node_dense0(cur, w1x, att_r):
    return pl.pallas_call(
        _node_dense0_body,
        grid=(_NG,),
        in_specs=[_blk((_NB, D)), _full((D, D)), _full((1, D))],
        out_specs=[_blk((_NB, D)), _blk((1, 4, 128))],
        out_shape=[
            jax.ShapeDtypeStruct((NP, D), jnp.float32),
            jax.ShapeDtypeStruct((_NG, 4, 128), jnp.float32),
        ],
    )(cur, w1x, att_r.reshape(1, D))


def _node_dense(spart, w2, bias, w1x, att_r):
    return pl.pallas_call(
        _node_dense_body,
        grid=(_NG,),
        in_specs=[
            pl.BlockSpec((NC, _NB, HD), lambda i: (0, i, 0)),
            _full((D, D)), _full((1, D)), _full((D, D)), _full((1, D)),
        ],
        out_specs=[_blk((_NB, D)), _blk((_NB, D)), _blk((1, 4, 128))],
        out_shape=[
            jax.ShapeDtypeStruct((NP, D), jnp.float32),
            jax.ShapeDtypeStruct((NP, D), jnp.float32),
            jax.ShapeDtypeStruct((_NG, 4, 128), jnp.float32),
        ],
    )(spart, w2, bias.reshape(1, D), w1x, att_r.reshape(1, D))


def _edge_dense(g, ea8, w1e8, att_l):
    return pl.pallas_call(
        _edge_dense_body,
        grid=(_EG,),
        in_specs=[_blk((_EB, D)), _blk((_EB, 8)), _full((8, D)), _full((1, D))],
        out_specs=[_blk((_EB, D)), _blk((1, 4, 128))],
        out_shape=[
            jax.ShapeDtypeStruct((E, D), jnp.float32),
            jax.ShapeDtypeStruct((_EG, 4, 128), jnp.float32),
        ],
    )(g, ea8, w1e8, att_l.reshape(1, D))


def _final(spart, w2, bias, h, c1, f1h, f1c1, f1c2, f1b, f2w, f2b):
    return pl.pallas_call(
        _final_body,
        grid=(_NG,),
        in_specs=[
            pl.BlockSpec((NC, _NB, HD), lambda i: (0, i, 0)),
            _full((D, D)), _full((1, D)),
            _blk((_NB, D)), _blk((_NB, D)),
            _full((D, 128)), _full((D, 128)), _full((D, 128)),
            _full((1, 128)), _full((128, 128)), _full((1, 128)),
        ],
        out_specs=[_blk((1, 4, 128))],
        out_shape=[jax.ShapeDtypeStruct((_NG, 4, 128), jnp.float32)],
    )(spart, w2, bias.reshape(1, D), h, c1, f1h, f1c1, f1c2, f1b, f2w, f2b)


def kernel(x, edge_index, edge_attr, emb_table,
           att_l0, att_r0, lin1_W0, lin2_W0, bias0,
           att_l1, att_r1, lin1_W1, lin2_W1, bias1,
           fc1_W, fc1_b, fc2_W, fc2_b):
    src = edge_index[0]
    dst = edge_index[1]
    x_pad = jnp.pad(x, (0, NP - N))
    ea8 = jnp.pad(edge_attr, ((0, 0), (0, 1)))

    h0 = _emb_gather(x_pad, emb_table)

    def layer(cur_args, att_l, att_r, lin1_W, lin2_W, bias, first):
        w1x = lin1_W[:D]
        w1e8 = jnp.pad(lin1_W[D:], ((0, 1), (0, 0)))
        if first:
            cur = cur_args
            xl1, ar2 = _node_dense0(cur, w1x, att_r)
        else:
            spart, w2p, bp = cur_args
            cur, xl1, ar2 = _node_dense(spart, w2p, bp, w1x, att_r)
        ar = ar2.reshape(NP)
        g = _edge_gather(src, xl1)
        hh, aj2 = _edge_dense(g, ea8, w1e8, att_l)
        aj = aj2.reshape(E)
        a, mxp = _alpha_max(aj, dst, ar)
        ex, denpart = _exp_den(a, dst, mxp)
        spart = _aggregate(hh, ex, dst, denpart)
        return cur, spart

    _, spart0 = layer(h0, att_l0, att_r0, lin1_W0, lin2_W0, bias0, True)
    cur1, spart1 = layer((spart0, lin2_W0, bias0),
                         att_l1, att_r1, lin1_W1, lin2_W1, bias1, False)

    f1h = jnp.pad(fc1_W[:D], ((0, 0), (0, 108)))
    f1c1 = jnp.pad(fc1_W[D:2 * D], ((0, 0), (0, 108)))
    f1c2 = jnp.pad(fc1_W[2 * D:], ((0, 0), (0, 108)))
    f1b = jnp.pad(fc1_b, (0, 108)).reshape(1, 128)
    f2w = jnp.pad(fc2_W, ((0, 108), (0, 127)))
    f2b = jnp.pad(fc2_b, (0, 127)).reshape(1, 128)

    [out2] = _final(spart1, lin2_W1, bias1, h0, cur1,
                  f1h, f1c1, f1c2, f1b, f2w, f2b)
    return out2.reshape(NP)[:N]
